# Initial kernel scaffold; baseline (speedup 1.0000x reference)
import functools
import math

import jax
import jax.numpy as jnp
import numpy as np
from jax import lax
from jax.experimental import pallas as pl
from jax.experimental.pallas import tpu as pltpu
from jax.experimental.pallas import tpu_sc as plsc

M = 10000
E = 320000
D = 128
H = 8
HD = D // H
SCALE = 1.0 / math.sqrt(HD)

NC = 2
NS = 16
NW = NC * NS
EPW = E // NW
B = 80
NCHUNK = EPW // B
ROWS_PER_TILE = M // NS

_j = np.arange(D)
_s, _l = _j // 16, _j % 16
_h, _c = _l % 8, 2 * _s + (_l // 8)
_Q_COLS = _h * HD + _c
_K_COLS = _h * 2 * HD + _c
_V_COLS = _h * 2 * HD + HD + _c

_RB = 2000


def _proj_body(s_ref, wq_ref, bq_ref, wk_ref, bk_ref, wv_ref, bv_ref,
               q_ref, k_ref, v_ref):
    sb = s_ref[...]
    q_ref[...] = jnp.dot(sb, wq_ref[...], preferred_element_type=jnp.float32) + bq_ref[...]
    k_ref[...] = jnp.dot(sb, wk_ref[...], preferred_element_type=jnp.float32) + bk_ref[...]
    v_ref[...] = jnp.dot(sb, wv_ref[...], preferred_element_type=jnp.float32) + bv_ref[...]


def _project(s, wq, bq, wk, bk, wv, bv):
    grid = (M // _RB,)
    row_spec = pl.BlockSpec((_RB, D), lambda i: (i, 0))
    full_spec = pl.BlockSpec((D, D), lambda i: (0, 0))
    bias_spec = pl.BlockSpec((1, D), lambda i: (0, 0))
    return pl.pallas_call(
        _proj_body,
        grid=grid,
        in_specs=[row_spec, full_spec, bias_spec, full_spec, bias_spec,
                  full_spec, bias_spec],
        out_specs=[row_spec, row_spec, row_spec],
        out_shape=[jax.ShapeDtypeStruct((M, D), jnp.float32)] * 3,
    )(s, wq, bq.reshape(1, D), wk, bk.reshape(1, D), wv, bv.reshape(1, D))


def _rot8(x):
    perm = (lax.iota(jnp.int32, 16) + 8) & 15
    dn = lax.GatherDimensionNumbers(
        offset_dims=(), collapsed_slice_dims=(0,), start_index_map=(0,))
    return lax.gather(x, perm[:, None], dn, (1,),
                      mode=lax.GatherScatterMode.PROMISE_IN_BOUNDS)


def _edge_kernel_body(qt_hbm, kt_hbm, vt_hbm, ei_hbm, num_hbm, den_hbm,
                      tgt_all, src_all, qv, kv, vv, wv, pv, znum, zden,
                      num_acc, den_acc, sem):
    cid = lax.axis_index("c")
    sid = lax.axis_index("s")
    wid = cid * NS + sid

    @pl.loop(0, 125)
    def _zfill(r):
        for t in range(D // 16):
            znum[r, pl.ds(t * 16, 16)] = jnp.zeros((16,), jnp.float32)
        zden[r, pl.ds(0, 16)] = jnp.zeros((16,), jnp.float32)

    @pl.loop(0, ROWS_PER_TILE // 125)
    def _zcopy(i):
        base = sid * ROWS_PER_TILE + i * 125
        pltpu.sync_copy(znum, num_acc.at[pl.ds(base, 125)])
        pltpu.sync_copy(zden, den_acc.at[pl.ds(base, 125)])

    pltpu.sync_copy(ei_hbm.at[1, wid], tgt_all)
    pltpu.sync_copy(ei_hbm.at[0, wid], src_all)

    plsc.subcore_barrier()

    @pl.loop(0, NCHUNK)
    def _chunk(ck):
        cq = pltpu.async_copy(qt_hbm.at[tgt_all.at[ck]], qv, sem)
        ckk = pltpu.async_copy(kt_hbm.at[src_all.at[ck]], kv, sem)
        cv = pltpu.async_copy(vt_hbm.at[src_all.at[ck]], vv, sem)
        cq.wait()
        ckk.wait()
        cv.wait()

        @pl.loop(0, B)
        def _edge(e):
            acc = qv[e, pl.ds(0, 16)] * kv[e, pl.ds(0, 16)]
            for t in range(1, 8):
                acc = acc + qv[e, pl.ds(t * 16, 16)] * kv[e, pl.ds(t * 16, 16)]
            p = jnp.exp((acc + _rot8(acc)) * SCALE)
            pv[e, pl.ds(0, 16)] = p
            for t in range(8):
                wv[e, pl.ds(t * 16, 16)] = vv[e, pl.ds(t * 16, 16)] * p

        pltpu.sync_copy(wv, num_acc.at[tgt_all.at[ck]], add=True)
        pltpu.sync_copy(pv, den_acc.at[tgt_all.at[ck]], add=True)

    plsc.subcore_barrier()

    base = sid * ROWS_PER_TILE
    pltpu.sync_copy(num_acc.at[pl.ds(base, ROWS_PER_TILE)],
                    num_hbm.at[cid, pl.ds(base, ROWS_PER_TILE)])
    pltpu.sync_copy(den_acc.at[pl.ds(base, ROWS_PER_TILE)],
                    den_hbm.at[cid, pl.ds(base, ROWS_PER_TILE)])


def _edge_stage(qt, kt, vt, ei):
    mesh = plsc.VectorSubcoreMesh(core_axis_name="c", subcore_axis_name="s")
    kern = pl.kernel(
        _edge_kernel_body,
        out_type=[
            jax.ShapeDtypeStruct((NC, M, D), jnp.float32),
            jax.ShapeDtypeStruct((NC, M, 16), jnp.float32),
        ],
        mesh=mesh,
        scratch_types=[
            pltpu.VMEM((NCHUNK, B), jnp.int32),
            pltpu.VMEM((NCHUNK, B), jnp.int32),
            pltpu.VMEM((B, D), jnp.float32),
            pltpu.VMEM((B, D), jnp.float32),
            pltpu.VMEM((B, D), jnp.float32),
            pltpu.VMEM((B, D), jnp.float32),
            pltpu.VMEM((B, 16), jnp.float32),
            pltpu.VMEM((125, D), jnp.float32),
            pltpu.VMEM((125, 16), jnp.float32),
            pltpu.VMEM_SHARED((M, D), jnp.float32),
            pltpu.VMEM_SHARED((M, 16), jnp.float32),
            pltpu.SemaphoreType.DMA,
        ],
    )
    return kern(qt, kt, vt, ei)


def _final_body(n0_ref, n1_ref, d0_ref, d1_ref, wo_ref, bo_ref, o_ref):
    numer = n0_ref[...] + n1_ref[...]
    den = d0_ref[...] + d1_ref[...]
    den_t = jnp.tile(den[:, :H], (1, D // H))
    o = numer / (den_t + 1e-16)
    o_ref[...] = jnp.dot(o, wo_ref[...], preferred_element_type=jnp.float32) + bo_ref[...]


def _finalize(num, den, wo, bo):
    grid = (M // _RB,)
    row_spec = pl.BlockSpec((_RB, D), lambda i: (i, 0))
    den_spec = pl.BlockSpec((_RB, 16), lambda i: (i, 0))
    full_spec = pl.BlockSpec((D, D), lambda i: (0, 0))
    bias_spec = pl.BlockSpec((1, D), lambda i: (0, 0))
    return pl.pallas_call(
        _final_body,
        grid=grid,
        in_specs=[row_spec, row_spec, den_spec, den_spec, full_spec, bias_spec],
        out_specs=pl.BlockSpec((_RB, D), lambda i: (i, 0)),
        out_shape=jax.ShapeDtypeStruct((M, D), jnp.float32),
    )(num[0], num[1], den[0], den[1], wo, bo.reshape(1, D))


def kernel(s, edge_index, Wq, bq, Wkv, bkv, Wo, bo):
    wq = Wq[:, _Q_COLS]
    bq_p = bq[_Q_COLS]
    wk = Wkv[:, _K_COLS]
    bk_p = bkv[_K_COLS]
    wv = Wkv[:, _V_COLS]
    bv_p = bkv[_V_COLS]
    wo_p = Wo[_Q_COLS, :]

    qt, kt, vt = _project(s, wq, bq_p, wk, bk_p, wv, bv_p)
    ei = edge_index.reshape(2, NW, NCHUNK, B)
    num, den = _edge_stage(qt, kt, vt, ei)
    return _finalize(num, den, wo_p, bo)


# speedup vs baseline: 35.7058x; 35.7058x over previous
import math

import jax
import jax.numpy as jnp
import numpy as np
from jax import lax
from jax.experimental import pallas as pl
from jax.experimental.pallas import tpu as pltpu
from jax.experimental.pallas import tpu_sc as plsc

M = 10000
E = 320000
D = 128
H = 8
HD = D // H
SCALE = 1.0 / math.sqrt(HD)

NC = 2
NS = 16
EPT = E // NS
B = 80
NCHUNK = EPT // B
MP = 10240
ROWS_PER_TILE = MP // NS
DH = D // NC

_j = np.arange(DH)
_s, _l = _j // 16, _j % 16
_hl, _c = _l % 4, 4 * _s + (_l // 4)


def _half_cols(g, base_of_h, off):
    h = 4 * g + _hl
    return h * base_of_h + off + _c


_Q_COLS = [_half_cols(g, HD, 0) for g in range(NC)]
_K_COLS = [_half_cols(g, 2 * HD, 0) for g in range(NC)]
_V_COLS = [_half_cols(g, 2 * HD, HD) for g in range(NC)]
_O_ROWS = np.concatenate([_Q_COLS[0], _Q_COLS[1]])

_RB = 2000


def _proj_body(s_ref, w_ref, b_ref, q_ref):
    q_ref[0] = (jnp.dot(s_ref[...], w_ref[0],
                        preferred_element_type=jnp.float32) + b_ref[0])


def _project(s, w, b):
    grid = (NC, M // _RB)
    return pl.pallas_call(
        _proj_body,
        grid=grid,
        in_specs=[
            pl.BlockSpec((_RB, D), lambda g, i: (i, 0)),
            pl.BlockSpec((1, D, 3 * DH), lambda g, i: (g, 0, 0)),
            pl.BlockSpec((1, 1, 3 * DH), lambda g, i: (g, 0, 0)),
        ],
        out_specs=pl.BlockSpec((1, _RB, 3 * DH), lambda g, i: (g, i, 0)),
        out_shape=jax.ShapeDtypeStruct((NC, M, 3 * DH), jnp.float32),
    )(s, w, b)


def _rot(x, r):
    perm = (lax.iota(jnp.int32, 16) + r) & 15
    dn = lax.GatherDimensionNumbers(
        offset_dims=(), collapsed_slice_dims=(0,), start_index_map=(0,))
    return lax.gather(x, perm[:, None], dn, (1,),
                      mode=lax.GatherScatterMode.PROMISE_IN_BOUNDS)


def _edge_kernel_body(qt_hbm, kt_hbm, vt_hbm, ei_hbm, num_hbm, den_hbm,
                      tgt_r, src_r, qv, kv, vv, wv, pv, num_acc, den_acc, sem):
    cid = lax.axis_index("c")
    sid = lax.axis_index("s")

    @pl.loop(0, B)
    def _zfill(r):
        for t in range(DH // 16):
            wv[r, pl.ds(t * 16, 16)] = jnp.zeros((16,), jnp.float32)
        pv[r, pl.ds(0, 16)] = jnp.zeros((16,), jnp.float32)

    @pl.loop(0, ROWS_PER_TILE // B)
    def _zcopy(i):
        base = sid * ROWS_PER_TILE + i * B
        pltpu.sync_copy(wv, num_acc.at[pl.ds(base, B)])
        pltpu.sync_copy(pv, den_acc.at[pl.ds(base, B)])

    plsc.subcore_barrier()

    @pl.loop(0, NCHUNK)
    def _chunk(ck):
        slot = ck & 1
        pltpu.sync_copy(ei_hbm.at[1, sid, ck], tgt_r.at[slot])
        pltpu.sync_copy(ei_hbm.at[0, sid, ck], src_r.at[slot])
        cq = pltpu.async_copy(qt_hbm.at[cid].at[tgt_r.at[slot]], qv, sem)
        ckk = pltpu.async_copy(kt_hbm.at[cid].at[src_r.at[slot]], kv, sem)
        cv = pltpu.async_copy(vt_hbm.at[cid].at[src_r.at[slot]], vv, sem)
        cq.wait()
        ckk.wait()
        cv.wait()

        @pl.loop(0, B)
        def _edge(e):
            acc = qv[e, pl.ds(0, 16)] * kv[e, pl.ds(0, 16)]
            for t in range(1, DH // 16):
                acc = acc + qv[e, pl.ds(t * 16, 16)] * kv[e, pl.ds(t * 16, 16)]
            t2 = acc + _rot(acc, 8)
            p = jnp.exp((t2 + _rot(t2, 4)) * SCALE)
            pv[e, pl.ds(0, 16)] = p
            for t in range(DH // 16):
                wv[e, pl.ds(t * 16, 16)] = vv[e, pl.ds(t * 16, 16)] * p

        pltpu.sync_copy(wv, num_acc.at[tgt_r.at[slot]], add=True)
        pltpu.sync_copy(pv, den_acc.at[tgt_r.at[slot]], add=True)

    plsc.subcore_barrier()

    base = sid * ROWS_PER_TILE
    pltpu.sync_copy(num_acc.at[pl.ds(base, ROWS_PER_TILE)],
                    num_hbm.at[cid, pl.ds(base, ROWS_PER_TILE)])
    pltpu.sync_copy(den_acc.at[pl.ds(base, ROWS_PER_TILE)],
                    den_hbm.at[cid, pl.ds(base, ROWS_PER_TILE)])


def _edge_stage(qt, kt, vt, ei):
    mesh = plsc.VectorSubcoreMesh(core_axis_name="c", subcore_axis_name="s")
    kern = pl.kernel(
        _edge_kernel_body,
        out_type=[
            jax.ShapeDtypeStruct((NC, MP, DH), jnp.float32),
            jax.ShapeDtypeStruct((NC, MP, 16), jnp.float32),
        ],
        mesh=mesh,
        scratch_types=[
            pltpu.VMEM((2, B), jnp.int32),
            pltpu.VMEM((2, B), jnp.int32),
            pltpu.VMEM((B, DH), jnp.float32),
            pltpu.VMEM((B, DH), jnp.float32),
            pltpu.VMEM((B, DH), jnp.float32),
            pltpu.VMEM((B, DH), jnp.float32),
            pltpu.VMEM((B, 16), jnp.float32),
            pltpu.VMEM_SHARED((MP, DH), jnp.float32),
            pltpu.VMEM_SHARED((MP, 16), jnp.float32),
            pltpu.SemaphoreType.DMA,
        ],
        compiler_params=pltpu.CompilerParams(use_tc_tiling_on_sc=False),
    )
    return kern(qt, kt, vt, ei)


def _final_body(n0_ref, n1_ref, d0_ref, d1_ref, wo_ref, bo_ref, o_ref):
    d0 = jnp.tile(d0_ref[0][:, :4], (1, 16))
    d1 = jnp.tile(d1_ref[0][:, :4], (1, 16))
    o0 = n0_ref[0] / (d0 + 1e-16)
    o1 = n1_ref[0] / (d1 + 1e-16)
    o_ref[...] = (jnp.dot(o0, wo_ref[:DH], preferred_element_type=jnp.float32)
                  + jnp.dot(o1, wo_ref[DH:], preferred_element_type=jnp.float32)
                  + bo_ref[...])


def _finalize(num, den, wo, bo):
    grid = (M // _RB,)
    return pl.pallas_call(
        _final_body,
        grid=grid,
        in_specs=[
            pl.BlockSpec((1, _RB, DH), lambda i: (0, i, 0)),
            pl.BlockSpec((1, _RB, DH), lambda i: (1, i, 0)),
            pl.BlockSpec((1, _RB, 16), lambda i: (0, i, 0)),
            pl.BlockSpec((1, _RB, 16), lambda i: (1, i, 0)),
            pl.BlockSpec((D, D), lambda i: (0, 0)),
            pl.BlockSpec((1, D), lambda i: (0, 0)),
        ],
        out_specs=pl.BlockSpec((_RB, D), lambda i: (i, 0)),
        out_shape=jax.ShapeDtypeStruct((M, D), jnp.float32),
    )(num, num, den, den, wo, bo.reshape(1, D))


def kernel(s, edge_index, Wq, bq, Wkv, bkv, Wo, bo):
    w = jnp.stack([
        jnp.concatenate([Wq[:, _Q_COLS[g]], Wkv[:, _K_COLS[g]],
                         Wkv[:, _V_COLS[g]]], axis=1)
        for g in range(NC)])
    b = jnp.stack([
        jnp.concatenate([bq[_Q_COLS[g]], bkv[_K_COLS[g]], bkv[_V_COLS[g]]])
        for g in range(NC)]).reshape(NC, 1, 3 * DH)
    wo_p = Wo[_O_ROWS, :]

    qkv = _project(s, w, b)
    qt = qkv[:, :, 0:DH]
    kt = qkv[:, :, DH:2 * DH]
    vt = qkv[:, :, 2 * DH:3 * DH]
    ei = edge_index.reshape(2, NS, NCHUNK, B)
    num, den = _edge_stage(qt, kt, vt, ei)
    return _finalize(num, den, wo_p, bo)


# speedup vs baseline: 52.0044x; 1.4565x over previous
import math

import jax
import jax.numpy as jnp
import numpy as np
from jax import lax
from jax.experimental import pallas as pl
from jax.experimental.pallas import tpu as pltpu
from jax.experimental.pallas import tpu_sc as plsc

M = 10000
E = 320000
D = 128
H = 8
HD = D // H
SCALE = 1.0 / math.sqrt(HD)

NC = 2
NS = 16
EPT = E // NS
B = 80
NCHUNK = EPT // B
MP = 10240
ROWS_PER_TILE = MP // NS
DH = D // NC

_j = np.arange(DH)
_s, _l = _j // 16, _j % 16
_hl, _c = _l % 4, 4 * _s + (_l // 4)


def _half_cols(g, base_of_h, off):
    h = 4 * g + _hl
    return h * base_of_h + off + _c


_Q_COLS = [_half_cols(g, HD, 0) for g in range(NC)]
_K_COLS = [_half_cols(g, 2 * HD, 0) for g in range(NC)]
_V_COLS = [_half_cols(g, 2 * HD, HD) for g in range(NC)]
_O_ROWS = np.concatenate([_Q_COLS[0], _Q_COLS[1]])

_RB = 2000


def _proj_body(s_ref, w_ref, b_ref, q_ref):
    q_ref[0] = (jnp.dot(s_ref[...], w_ref[0],
                        preferred_element_type=jnp.float32) + b_ref[0])


def _project(s, w, b):
    grid = (NC, M // _RB)
    return pl.pallas_call(
        _proj_body,
        grid=grid,
        in_specs=[
            pl.BlockSpec((_RB, D), lambda g, i: (i, 0)),
            pl.BlockSpec((1, D, 3 * DH), lambda g, i: (g, 0, 0)),
            pl.BlockSpec((1, 1, 3 * DH), lambda g, i: (g, 0, 0)),
        ],
        out_specs=pl.BlockSpec((1, _RB, 3 * DH), lambda g, i: (g, i, 0)),
        out_shape=jax.ShapeDtypeStruct((NC, M, 3 * DH), jnp.float32),
    )(s, w, b)


def _rot(x, r):
    perm = (lax.iota(jnp.int32, 16) + r) & 15
    dn = lax.GatherDimensionNumbers(
        offset_dims=(), collapsed_slice_dims=(0,), start_index_map=(0,))
    return lax.gather(x, perm[:, None], dn, (1,),
                      mode=lax.GatherScatterMode.PROMISE_IN_BOUNDS)


def _edge_kernel_body(qt_hbm, kt_hbm, vt_hbm, ei_hbm, num_hbm, den_hbm,
                      tgt_r, src_r, qv, kv, vv, wv, pv, num_acc, den_acc,
                      gsem0, gsem1, ssem0, ssem1):
    cid = lax.axis_index("c")
    sid = lax.axis_index("s")
    gsems = (gsem0, gsem1)
    ssems = (ssem0, ssem1)

    @pl.loop(0, B)
    def _zfill(r):
        for t in range(DH // 16):
            wv[0, r, pl.ds(t * 16, 16)] = jnp.zeros((16,), jnp.float32)
        pv[0, r, pl.ds(0, 16)] = jnp.zeros((16,), jnp.float32)

    @pl.loop(0, ROWS_PER_TILE // B)
    def _zcopy(i):
        base = sid * ROWS_PER_TILE + i * B
        pltpu.sync_copy(wv.at[0], num_acc.at[pl.ds(base, B)])
        pltpu.sync_copy(pv.at[0], den_acc.at[pl.ds(base, B)])

    plsc.subcore_barrier()

    def load_idx(ck):
        pltpu.sync_copy(ei_hbm.at[1, sid, ck], tgt_r.at[ck & 3])
        pltpu.sync_copy(ei_hbm.at[0, sid, ck], src_r.at[ck & 3])

    def start_gathers(ck, b):
        pltpu.async_copy(qt_hbm.at[cid].at[tgt_r.at[ck & 3]], qv.at[b], gsems[b])
        pltpu.async_copy(kt_hbm.at[cid].at[src_r.at[ck & 3]], kv.at[b], gsems[b])
        pltpu.async_copy(vt_hbm.at[cid].at[src_r.at[ck & 3]], vv.at[b], gsems[b])

    def drain_gathers(ck, b):
        pltpu.make_async_copy(qt_hbm.at[cid].at[tgt_r.at[ck & 3]], qv.at[b], gsems[b]).wait()
        pltpu.make_async_copy(kt_hbm.at[cid].at[src_r.at[ck & 3]], kv.at[b], gsems[b]).wait()
        pltpu.make_async_copy(vt_hbm.at[cid].at[src_r.at[ck & 3]], vv.at[b], gsems[b]).wait()

    def drain_scatters(ck, b):
        pltpu.make_async_copy(wv.at[b], num_acc.at[tgt_r.at[ck & 3]], ssems[b]).wait()
        pltpu.make_async_copy(pv.at[b], den_acc.at[tgt_r.at[ck & 3]], ssems[b]).wait()

    load_idx(0)
    start_gathers(0, 0)

    @pl.loop(0, NCHUNK, step=2)
    def _chunk2(ck0):
        for b in range(2):
            ck = ck0 + b

            @pl.when(ck + 1 < NCHUNK)
            def _prefetch():
                load_idx(ck + 1)
                start_gathers(ck + 1, 1 - b)

            drain_gathers(ck, b)

            @pl.when(ck >= 2)
            def _drain_old_scatter():
                drain_scatters(ck - 2, b)

            @pl.loop(0, B)
            def _edge(e):
                acc = qv[b, e, pl.ds(0, 16)] * kv[b, e, pl.ds(0, 16)]
                for t in range(1, DH // 16):
                    acc = acc + qv[b, e, pl.ds(t * 16, 16)] * kv[b, e, pl.ds(t * 16, 16)]
                t2 = acc + _rot(acc, 8)
                p = jnp.exp((t2 + _rot(t2, 4)) * SCALE)
                pv[b, e, pl.ds(0, 16)] = p
                for t in range(DH // 16):
                    wv[b, e, pl.ds(t * 16, 16)] = vv[b, e, pl.ds(t * 16, 16)] * p

            pltpu.async_copy(wv.at[b], num_acc.at[tgt_r.at[ck & 3]], ssems[b], add=True)
            pltpu.async_copy(pv.at[b], den_acc.at[tgt_r.at[ck & 3]], ssems[b], add=True)

    drain_scatters(NCHUNK - 2, 0)
    drain_scatters(NCHUNK - 1, 1)

    plsc.subcore_barrier()

    base = sid * ROWS_PER_TILE
    pltpu.sync_copy(num_acc.at[pl.ds(base, ROWS_PER_TILE)],
                    num_hbm.at[cid, pl.ds(base, ROWS_PER_TILE)])
    pltpu.sync_copy(den_acc.at[pl.ds(base, ROWS_PER_TILE)],
                    den_hbm.at[cid, pl.ds(base, ROWS_PER_TILE)])


def _edge_stage(qt, kt, vt, ei):
    mesh = plsc.VectorSubcoreMesh(core_axis_name="c", subcore_axis_name="s")
    kern = pl.kernel(
        _edge_kernel_body,
        out_type=[
            jax.ShapeDtypeStruct((NC, MP, DH), jnp.float32),
            jax.ShapeDtypeStruct((NC, MP, 16), jnp.float32),
        ],
        mesh=mesh,
        scratch_types=[
            pltpu.VMEM((4, B), jnp.int32),
            pltpu.VMEM((4, B), jnp.int32),
            pltpu.VMEM((2, B, DH), jnp.float32),
            pltpu.VMEM((2, B, DH), jnp.float32),
            pltpu.VMEM((2, B, DH), jnp.float32),
            pltpu.VMEM((2, B, DH), jnp.float32),
            pltpu.VMEM((2, B, 16), jnp.float32),
            pltpu.VMEM_SHARED((MP, DH), jnp.float32),
            pltpu.VMEM_SHARED((MP, 16), jnp.float32),
            pltpu.SemaphoreType.DMA,
            pltpu.SemaphoreType.DMA,
            pltpu.SemaphoreType.DMA,
            pltpu.SemaphoreType.DMA,
        ],
        compiler_params=pltpu.CompilerParams(use_tc_tiling_on_sc=False),
    )
    return kern(qt, kt, vt, ei)


def _final_body(n0_ref, n1_ref, d0_ref, d1_ref, wo_ref, bo_ref, o_ref):
    d0 = jnp.tile(d0_ref[0][:, :4], (1, 16))
    d1 = jnp.tile(d1_ref[0][:, :4], (1, 16))
    o0 = n0_ref[0] / (d0 + 1e-16)
    o1 = n1_ref[0] / (d1 + 1e-16)
    o_ref[...] = (jnp.dot(o0, wo_ref[:DH], preferred_element_type=jnp.float32)
                  + jnp.dot(o1, wo_ref[DH:], preferred_element_type=jnp.float32)
                  + bo_ref[...])


def _finalize(num, den, wo, bo):
    grid = (M // _RB,)
    return pl.pallas_call(
        _final_body,
        grid=grid,
        in_specs=[
            pl.BlockSpec((1, _RB, DH), lambda i: (0, i, 0)),
            pl.BlockSpec((1, _RB, DH), lambda i: (1, i, 0)),
            pl.BlockSpec((1, _RB, 16), lambda i: (0, i, 0)),
            pl.BlockSpec((1, _RB, 16), lambda i: (1, i, 0)),
            pl.BlockSpec((D, D), lambda i: (0, 0)),
            pl.BlockSpec((1, D), lambda i: (0, 0)),
        ],
        out_specs=pl.BlockSpec((_RB, D), lambda i: (i, 0)),
        out_shape=jax.ShapeDtypeStruct((M, D), jnp.float32),
    )(num, num, den, den, wo, bo.reshape(1, D))


def kernel(s, edge_index, Wq, bq, Wkv, bkv, Wo, bo):
    w = jnp.stack([
        jnp.concatenate([Wq[:, _Q_COLS[g]], Wkv[:, _K_COLS[g]],
                         Wkv[:, _V_COLS[g]]], axis=1)
        for g in range(NC)])
    b = jnp.stack([
        jnp.concatenate([bq[_Q_COLS[g]], bkv[_K_COLS[g]], bkv[_V_COLS[g]]])
        for g in range(NC)]).reshape(NC, 1, 3 * DH)
    wo_p = Wo[_O_ROWS, :]

    qkv = _project(s, w, b)
    qt = qkv[:, :, 0:DH]
    kt = qkv[:, :, DH:2 * DH]
    vt = qkv[:, :, 2 * DH:3 * DH]
    ei = edge_index.reshape(2, NS, NCHUNK, B)
    num, den = _edge_stage(qt, kt, vt, ei)
    return _finalize(num, den, wo_p, bo)


# speedup vs baseline: 84.0630x; 1.6165x over previous
import math

import jax
import jax.numpy as jnp
import numpy as np
from jax import lax
from jax.experimental import pallas as pl
from jax.experimental.pallas import tpu as pltpu
from jax.experimental.pallas import tpu_sc as plsc

M = 10000
E = 320000
D = 128
H = 8
HD = D // H
SCALE = 1.0 / math.sqrt(HD)

NC = 2
NS = 16
EPT = E // NS
B = 80
NCHUNK = EPT // B
MP = 10240
ROWS_PER_TILE = MP // NS
DH = D // NC

_j = np.arange(DH)
_s, _l = _j // 16, _j % 16
_hl, _c = _l % 4, 4 * _s + (_l // 4)


def _half_cols(g, base_of_h, off):
    h = 4 * g + _hl
    return h * base_of_h + off + _c


_Q_COLS = [_half_cols(g, HD, 0) for g in range(NC)]
_K_COLS = [_half_cols(g, 2 * HD, 0) for g in range(NC)]
_V_COLS = [_half_cols(g, 2 * HD, HD) for g in range(NC)]
_O_ROWS = np.concatenate([_Q_COLS[0], _Q_COLS[1]])

_RB = 2000


def _proj_body(s_ref, w_ref, b_ref, q_ref, kv_ref):
    out = (jnp.dot(s_ref[...], w_ref[0],
                   preferred_element_type=jnp.float32) + b_ref[0])
    q_ref[0] = out[:, :DH]
    kv_ref[0] = out[:, DH:]


def _project(s, w, b):
    grid = (NC, M // _RB)
    return pl.pallas_call(
        _proj_body,
        grid=grid,
        in_specs=[
            pl.BlockSpec((_RB, D), lambda g, i: (i, 0)),
            pl.BlockSpec((1, D, 3 * DH), lambda g, i: (g, 0, 0)),
            pl.BlockSpec((1, 1, 3 * DH), lambda g, i: (g, 0, 0)),
        ],
        out_specs=[pl.BlockSpec((1, _RB, DH), lambda g, i: (g, i, 0)),
                   pl.BlockSpec((1, _RB, 2 * DH), lambda g, i: (g, i, 0))],
        out_shape=[jax.ShapeDtypeStruct((NC, M, DH), jnp.float32),
                   jax.ShapeDtypeStruct((NC, M, 2 * DH), jnp.float32)],
    )(s, w, b)


def _rot(x, r):
    perm = (lax.iota(jnp.int32, 16) + r) & 15
    dn = lax.GatherDimensionNumbers(
        offset_dims=(), collapsed_slice_dims=(0,), start_index_map=(0,))
    return lax.gather(x, perm[:, None], dn, (1,),
                      mode=lax.GatherScatterMode.PROMISE_IN_BOUNDS)


WOUT = DH + 16


def _edge_kernel_body(qt_hbm, kvt_hbm, ei_hbm, acc_hbm,
                      tgt_r, src_r, qv, kvv, wv, acc,
                      gsem0, gsem1, ssem0, ssem1):
    cid = lax.axis_index("c")
    sid = lax.axis_index("s")
    gsems = (gsem0, gsem1)
    ssems = (ssem0, ssem1)

    @pl.loop(0, B)
    def _zfill(r):
        for t in range(WOUT // 16):
            wv[0, r, pl.ds(t * 16, 16)] = jnp.zeros((16,), jnp.float32)

    @pl.loop(0, ROWS_PER_TILE // B)
    def _zcopy(i):
        base = sid * ROWS_PER_TILE + i * B
        pltpu.sync_copy(wv.at[0], acc.at[pl.ds(base, B)])

    plsc.subcore_barrier()

    def load_idx(ck):
        pltpu.sync_copy(ei_hbm.at[1, sid, ck], tgt_r.at[ck & 3])
        pltpu.sync_copy(ei_hbm.at[0, sid, ck], src_r.at[ck & 3])

    def start_gathers(ck, b):
        pltpu.async_copy(qt_hbm.at[cid].at[tgt_r.at[ck & 3]], qv.at[b], gsems[b])
        pltpu.async_copy(kvt_hbm.at[cid].at[src_r.at[ck & 3]], kvv.at[b], gsems[b])

    def drain_gathers(ck, b):
        pltpu.make_async_copy(qt_hbm.at[cid].at[tgt_r.at[ck & 3]], qv.at[b], gsems[b]).wait()
        pltpu.make_async_copy(kvt_hbm.at[cid].at[src_r.at[ck & 3]], kvv.at[b], gsems[b]).wait()

    def drain_scatter(ck, b):
        pltpu.make_async_copy(wv.at[b], acc.at[tgt_r.at[ck & 3]], ssems[b]).wait()

    load_idx(0)
    start_gathers(0, 0)

    @pl.loop(0, NCHUNK, step=2)
    def _chunk2(ck0):
        for b in range(2):
            ck = ck0 + b

            @pl.when(ck + 1 < NCHUNK)
            def _prefetch():
                load_idx(ck + 1)
                start_gathers(ck + 1, 1 - b)

            drain_gathers(ck, b)

            @pl.when(ck >= 2)
            def _drain_old_scatter():
                drain_scatter(ck - 2, b)

            @plsc.parallel_loop(0, B, 1, unroll=2)
            def _edge(e):
                acc16 = qv[b, e, pl.ds(0, 16)] * kvv[b, e, pl.ds(0, 16)]
                for t in range(1, DH // 16):
                    acc16 = acc16 + qv[b, e, pl.ds(t * 16, 16)] * kvv[b, e, pl.ds(t * 16, 16)]
                t2 = acc16 + _rot(acc16, 8)
                p = jnp.exp((t2 + _rot(t2, 4)) * SCALE)
                wv[b, e, pl.ds(DH, 16)] = p
                for t in range(DH // 16):
                    wv[b, e, pl.ds(t * 16, 16)] = kvv[b, e, pl.ds(DH + t * 16, 16)] * p

            pltpu.async_copy(wv.at[b], acc.at[tgt_r.at[ck & 3]], ssems[b], add=True)

    drain_scatter(NCHUNK - 2, 0)
    drain_scatter(NCHUNK - 1, 1)

    plsc.subcore_barrier()

    base = sid * ROWS_PER_TILE
    pltpu.sync_copy(acc.at[pl.ds(base, ROWS_PER_TILE)],
                    acc_hbm.at[cid, pl.ds(base, ROWS_PER_TILE)])


def _edge_stage(qt, kvt, ei):
    mesh = plsc.VectorSubcoreMesh(core_axis_name="c", subcore_axis_name="s")
    kern = pl.kernel(
        _edge_kernel_body,
        out_type=jax.ShapeDtypeStruct((NC, MP, WOUT), jnp.float32),
        mesh=mesh,
        scratch_types=[
            pltpu.VMEM((4, B), jnp.int32),
            pltpu.VMEM((4, B), jnp.int32),
            pltpu.VMEM((2, B, DH), jnp.float32),
            pltpu.VMEM((2, B, 2 * DH), jnp.float32),
            pltpu.VMEM((2, B, WOUT), jnp.float32),
            pltpu.VMEM_SHARED((MP, WOUT), jnp.float32),
            pltpu.SemaphoreType.DMA,
            pltpu.SemaphoreType.DMA,
            pltpu.SemaphoreType.DMA,
            pltpu.SemaphoreType.DMA,
        ],
        compiler_params=pltpu.CompilerParams(use_tc_tiling_on_sc=False),
    )
    return kern(qt, kvt, ei)


def _final_body(a0_ref, a1_ref, wo_ref, bo_ref, o_ref):
    a0 = a0_ref[0]
    a1 = a1_ref[0]
    d0 = jnp.tile(a0[:, DH:DH + 4], (1, 16))
    d1 = jnp.tile(a1[:, DH:DH + 4], (1, 16))
    o0 = a0[:, :DH] / (d0 + 1e-16)
    o1 = a1[:, :DH] / (d1 + 1e-16)
    o_ref[...] = (jnp.dot(o0, wo_ref[:DH], preferred_element_type=jnp.float32)
                  + jnp.dot(o1, wo_ref[DH:], preferred_element_type=jnp.float32)
                  + bo_ref[...])


def _finalize(acc, wo, bo):
    grid = (M // _RB,)
    return pl.pallas_call(
        _final_body,
        grid=grid,
        in_specs=[
            pl.BlockSpec((1, _RB, WOUT), lambda i: (0, i, 0)),
            pl.BlockSpec((1, _RB, WOUT), lambda i: (1, i, 0)),
            pl.BlockSpec((D, D), lambda i: (0, 0)),
            pl.BlockSpec((1, D), lambda i: (0, 0)),
        ],
        out_specs=pl.BlockSpec((_RB, D), lambda i: (i, 0)),
        out_shape=jax.ShapeDtypeStruct((M, D), jnp.float32),
    )(acc, acc, wo, bo.reshape(1, D))


def kernel(s, edge_index, Wq, bq, Wkv, bkv, Wo, bo):
    w = jnp.stack([
        jnp.concatenate([Wq[:, _Q_COLS[g]], Wkv[:, _K_COLS[g]],
                         Wkv[:, _V_COLS[g]]], axis=1)
        for g in range(NC)])
    b = jnp.stack([
        jnp.concatenate([bq[_Q_COLS[g]], bkv[_K_COLS[g]], bkv[_V_COLS[g]]])
        for g in range(NC)]).reshape(NC, 1, 3 * DH)
    wo_p = Wo[_O_ROWS, :]

    qt, kvt = _project(s, w, b)
    ei = edge_index.reshape(2, NS, NCHUNK, B)
    acc = _edge_stage(qt, kvt, ei)
    return _finalize(acc, wo_p, bo)


# speedup vs baseline: 87.3213x; 1.0388x over previous
import math

import jax
import jax.numpy as jnp
import numpy as np
from jax import lax
from jax.experimental import pallas as pl
from jax.experimental.pallas import tpu as pltpu
from jax.experimental.pallas import tpu_sc as plsc

M = 10000
E = 320000
D = 128
H = 8
HD = D // H
SCALE = 1.0 / math.sqrt(HD)

NC = 2
NS = 16
EPT = E // NS
B = 80
NCHUNK = EPT // B
MP = 10240
ROWS_PER_TILE = MP // NS
DH = D // NC

_m = np.arange(DH)
_t, _r = _m // 32, _m % 32
_i, _e = _r // 2, _r % 2
_hl = _i % 4
_c = 8 * _t + 2 * (_i // 4) + _e


def _half_cols(g, base_of_h, off):
    h = 4 * g + _hl
    return h * base_of_h + off + _c


_Q_COLS = [_half_cols(g, HD, 0) for g in range(NC)]
_K_COLS = [_half_cols(g, 2 * HD, 0) for g in range(NC)]
_V_COLS = [_half_cols(g, 2 * HD, HD) for g in range(NC)]

_Jw = np.arange(DH)
_u, _iw = _Jw // 16, _Jw % 16
_tw, _ew = _u // 2, _u % 2
_O_ROWS = np.concatenate([
    (4 * g + _iw % 4) * HD + (8 * _tw + 2 * (_iw // 4) + _ew)
    for g in range(NC)])

_RB = 2000


def _proj_body(s_ref, w_ref, b_ref, q_ref, k_ref, v_ref):
    out = (jnp.dot(s_ref[...], w_ref[0],
                   preferred_element_type=jnp.float32) + b_ref[0])
    out = out.astype(jnp.bfloat16)
    q_ref[0] = out[:, :DH]
    k_ref[0] = out[:, DH:2 * DH]
    v_ref[0] = out[:, 2 * DH:]


def _project(s, w, b):
    grid = (NC, M // _RB)
    out_spec = pl.BlockSpec((1, _RB, DH), lambda g, i: (g, i, 0))
    return pl.pallas_call(
        _proj_body,
        grid=grid,
        in_specs=[
            pl.BlockSpec((_RB, D), lambda g, i: (i, 0)),
            pl.BlockSpec((1, D, 3 * DH), lambda g, i: (g, 0, 0)),
            pl.BlockSpec((1, 1, 3 * DH), lambda g, i: (g, 0, 0)),
        ],
        out_specs=[out_spec, out_spec, out_spec],
        out_shape=[jax.ShapeDtypeStruct((NC, M, DH), jnp.bfloat16)] * 3,
    )(s, w, b)


def _rot(x, r):
    perm = (lax.iota(jnp.int32, 16) + r) & 15
    dn = lax.GatherDimensionNumbers(
        offset_dims=(), collapsed_slice_dims=(0,), start_index_map=(0,))
    return lax.gather(x, perm[:, None], dn, (1,),
                      mode=lax.GatherScatterMode.PROMISE_IN_BOUNDS)


WOUT = DH + 16


def _edge_kernel_body(qt_hbm, kt_hbm, vt_hbm, ei_hbm, acc_hbm,
                      tgt_r, src_r, qv, kv, vv, wv, acc,
                      gsem0, gsem1, ssem0, ssem1):
    cid = lax.axis_index("c")
    sid = lax.axis_index("s")
    gsems = (gsem0, gsem1)
    ssems = (ssem0, ssem1)

    @pl.loop(0, B)
    def _zfill(r):
        for t in range(WOUT // 16):
            wv[0, r, pl.ds(t * 16, 16)] = jnp.zeros((16,), jnp.float32)

    @pl.loop(0, ROWS_PER_TILE // B)
    def _zcopy(i):
        base = sid * ROWS_PER_TILE + i * B
        pltpu.sync_copy(wv.at[0], acc.at[pl.ds(base, B)])

    plsc.subcore_barrier()

    def load_idx(ck):
        pltpu.sync_copy(ei_hbm.at[1, sid, ck], tgt_r.at[ck & 3])
        pltpu.sync_copy(ei_hbm.at[0, sid, ck], src_r.at[ck & 3])

    def start_gathers(ck, b):
        pltpu.async_copy(qt_hbm.at[cid].at[tgt_r.at[ck & 3]], qv.at[b], gsems[b])
        pltpu.async_copy(kt_hbm.at[cid].at[src_r.at[ck & 3]], kv.at[b], gsems[b])
        pltpu.async_copy(vt_hbm.at[cid].at[src_r.at[ck & 3]], vv.at[b], gsems[b])

    def drain_gathers(ck, b):
        pltpu.make_async_copy(qt_hbm.at[cid].at[tgt_r.at[ck & 3]], qv.at[b], gsems[b]).wait()
        pltpu.make_async_copy(kt_hbm.at[cid].at[src_r.at[ck & 3]], kv.at[b], gsems[b]).wait()
        pltpu.make_async_copy(vt_hbm.at[cid].at[src_r.at[ck & 3]], vv.at[b], gsems[b]).wait()

    def drain_scatter(ck, b):
        pltpu.make_async_copy(wv.at[b], acc.at[tgt_r.at[ck & 3]], ssems[b]).wait()

    load_idx(0)
    start_gathers(0, 0)

    @pl.loop(0, NCHUNK, step=2)
    def _chunk2(ck0):
        for b in range(2):
            ck = ck0 + b

            @pl.when(ck + 1 < NCHUNK)
            def _prefetch():
                load_idx(ck + 1)
                start_gathers(ck + 1, 1 - b)

            drain_gathers(ck, b)

            @pl.when(ck >= 2)
            def _drain_old_scatter():
                drain_scatter(ck - 2, b)

            @plsc.parallel_loop(0, B, 1, unroll=2)
            def _edge(e):
                acc16 = None
                for t in range(DH // 32):
                    qa, qb = plsc.unpack(qv[b, e, pl.ds(t * 32, 32)],
                                         format=plsc.PackFormat.INTERLEAVED)
                    ka, kb = plsc.unpack(kv[b, e, pl.ds(t * 32, 32)],
                                         format=plsc.PackFormat.INTERLEAVED)
                    term = qa * ka + qb * kb
                    acc16 = term if acc16 is None else acc16 + term
                t2 = acc16 + _rot(acc16, 8)
                p = jnp.exp((t2 + _rot(t2, 4)) * SCALE)
                wv[b, e, pl.ds(DH, 16)] = p
                for t in range(DH // 32):
                    va, vb = plsc.unpack(vv[b, e, pl.ds(t * 32, 32)],
                                         format=plsc.PackFormat.INTERLEAVED)
                    wv[b, e, pl.ds(t * 32, 16)] = va * p
                    wv[b, e, pl.ds(t * 32 + 16, 16)] = vb * p

            pltpu.async_copy(wv.at[b], acc.at[tgt_r.at[ck & 3]], ssems[b], add=True)

    drain_scatter(NCHUNK - 2, 0)
    drain_scatter(NCHUNK - 1, 1)

    plsc.subcore_barrier()

    base = sid * ROWS_PER_TILE
    pltpu.sync_copy(acc.at[pl.ds(base, ROWS_PER_TILE)],
                    acc_hbm.at[cid, pl.ds(base, ROWS_PER_TILE)])


def _edge_stage(qt, kt, vt, ei):
    mesh = plsc.VectorSubcoreMesh(core_axis_name="c", subcore_axis_name="s")
    kern = pl.kernel(
        _edge_kernel_body,
        out_type=jax.ShapeDtypeStruct((NC, MP, WOUT), jnp.float32),
        mesh=mesh,
        scratch_types=[
            pltpu.VMEM((4, B), jnp.int32),
            pltpu.VMEM((4, B), jnp.int32),
            pltpu.VMEM((2, B, DH), jnp.bfloat16),
            pltpu.VMEM((2, B, DH), jnp.bfloat16),
            pltpu.VMEM((2, B, DH), jnp.bfloat16),
            pltpu.VMEM((2, B, WOUT), jnp.float32),
            pltpu.VMEM_SHARED((MP, WOUT), jnp.float32),
            pltpu.SemaphoreType.DMA,
            pltpu.SemaphoreType.DMA,
            pltpu.SemaphoreType.DMA,
            pltpu.SemaphoreType.DMA,
        ],
        compiler_params=pltpu.CompilerParams(use_tc_tiling_on_sc=False,
                                             needs_layout_passes=False),
    )
    return kern(qt, kt, vt, ei)


def _final_body(a0_ref, a1_ref, wo_ref, bo_ref, o_ref):
    a0 = a0_ref[0]
    a1 = a1_ref[0]
    d0 = jnp.tile(a0[:, DH:DH + 4], (1, 16))
    d1 = jnp.tile(a1[:, DH:DH + 4], (1, 16))
    o0 = a0[:, :DH] / (d0 + 1e-16)
    o1 = a1[:, :DH] / (d1 + 1e-16)
    o_ref[...] = (jnp.dot(o0, wo_ref[:DH], preferred_element_type=jnp.float32)
                  + jnp.dot(o1, wo_ref[DH:], preferred_element_type=jnp.float32)
                  + bo_ref[...])


def _finalize(acc, wo, bo):
    grid = (M // _RB,)
    return pl.pallas_call(
        _final_body,
        grid=grid,
        in_specs=[
            pl.BlockSpec((1, _RB, WOUT), lambda i: (0, i, 0)),
            pl.BlockSpec((1, _RB, WOUT), lambda i: (1, i, 0)),
            pl.BlockSpec((D, D), lambda i: (0, 0)),
            pl.BlockSpec((1, D), lambda i: (0, 0)),
        ],
        out_specs=pl.BlockSpec((_RB, D), lambda i: (i, 0)),
        out_shape=jax.ShapeDtypeStruct((M, D), jnp.float32),
    )(acc, acc, wo, bo.reshape(1, D))


def kernel(s, edge_index, Wq, bq, Wkv, bkv, Wo, bo):
    w = jnp.stack([
        jnp.concatenate([Wq[:, _Q_COLS[g]], Wkv[:, _K_COLS[g]],
                         Wkv[:, _V_COLS[g]]], axis=1)
        for g in range(NC)])
    b = jnp.stack([
        jnp.concatenate([bq[_Q_COLS[g]], bkv[_K_COLS[g]], bkv[_V_COLS[g]]])
        for g in range(NC)]).reshape(NC, 1, 3 * DH)
    wo_p = Wo[_O_ROWS, :]

    qt, kt, vt = _project(s, w, b)
    ei = edge_index.reshape(2, NS, NCHUNK, B)
    acc = _edge_stage(qt, kt, vt, ei)
    return _finalize(acc, wo_p, bo)


# speedup vs baseline: 126.7557x; 1.4516x over previous
import math

import jax
import jax.numpy as jnp
import numpy as np
from jax import lax
from jax.experimental import pallas as pl
from jax.experimental.pallas import tpu as pltpu
from jax.experimental.pallas import tpu_sc as plsc

M = 10000
E = 320000
D = 128
H = 8
HD = D // H
SCALE = 1.0 / math.sqrt(HD)

NC = 2
NS = 16
EPT = E // NS
B = 80
NCHUNK = EPT // B
SBC = 10
NSB = NCHUNK // SBC
MP = 10240
ROWS_PER_TILE = MP // NS
DH = D // NC

_m = np.arange(DH)
_t, _r = _m // 32, _m % 32
_i, _e = _r // 2, _r % 2
_hl = _i % 4
_c = 8 * _t + 2 * (_i // 4) + _e


def _half_cols(g, base_of_h, off):
    h = 4 * g + _hl
    return h * base_of_h + off + _c


_Q_COLS = [_half_cols(g, HD, 0) for g in range(NC)]
_K_COLS = [_half_cols(g, 2 * HD, 0) for g in range(NC)]
_V_COLS = [_half_cols(g, 2 * HD, HD) for g in range(NC)]

_Jw = np.arange(DH)
_u, _iw = _Jw // 16, _Jw % 16
_tw, _ew = _u // 2, _u % 2
_O_ROWS = np.concatenate([
    (4 * g + _iw % 4) * HD + (8 * _tw + 2 * (_iw // 4) + _ew)
    for g in range(NC)])

_RB = 2000


def _proj_body(s_ref, w_ref, b_ref, q_ref, k_ref, v_ref):
    out = (jnp.dot(s_ref[...], w_ref[0],
                   preferred_element_type=jnp.float32) + b_ref[0])
    out = out.astype(jnp.bfloat16)
    q_ref[0] = out[:, :DH]
    k_ref[0] = out[:, DH:2 * DH]
    v_ref[0] = out[:, 2 * DH:]


def _project(s, w, b):
    grid = (NC, M // _RB)
    out_spec = pl.BlockSpec((1, _RB, DH), lambda g, i: (g, i, 0))
    return pl.pallas_call(
        _proj_body,
        grid=grid,
        in_specs=[
            pl.BlockSpec((_RB, D), lambda g, i: (i, 0)),
            pl.BlockSpec((1, D, 3 * DH), lambda g, i: (g, 0, 0)),
            pl.BlockSpec((1, 1, 3 * DH), lambda g, i: (g, 0, 0)),
        ],
        out_specs=[out_spec, out_spec, out_spec],
        out_shape=[jax.ShapeDtypeStruct((NC, M, DH), jnp.bfloat16)] * 3,
    )(s, w, b)


def _rot(x, r):
    perm = (lax.iota(jnp.int32, 16) + r) & 15
    dn = lax.GatherDimensionNumbers(
        offset_dims=(), collapsed_slice_dims=(0,), start_index_map=(0,))
    return lax.gather(x, perm[:, None], dn, (1,),
                      mode=lax.GatherScatterMode.PROMISE_IN_BOUNDS)


WOUT = DH + 16


def _edge_kernel_body(qt_hbm, kt_hbm, vt_hbm, ei_hbm, acc_hbm,
                      tgt_r, src_r, qv, kv, vv, wv, acc,
                      gsem0, gsem1, ssem0, ssem1, isem):
    cid = lax.axis_index("c")
    sid = lax.axis_index("s")
    gsems = (gsem0, gsem1)
    ssems = (ssem0, ssem1)

    @pl.loop(0, B)
    def _zfill(r):
        for t in range(WOUT // 16):
            wv[0, r, pl.ds(t * 16, 16)] = jnp.zeros((16,), jnp.float32)

    @pl.loop(0, ROWS_PER_TILE // B)
    def _zcopy(i):
        base = sid * ROWS_PER_TILE + i * B
        pltpu.sync_copy(wv.at[0], acc.at[pl.ds(base, B)])

    plsc.subcore_barrier()

    def load_idx_sb_sync(sb):
        pltpu.sync_copy(ei_hbm.at[1, sid, pl.ds(sb * SBC, SBC)], tgt_r.at[sb & 1])
        pltpu.sync_copy(ei_hbm.at[0, sid, pl.ds(sb * SBC, SBC)], src_r.at[sb & 1])

    def start_idx_sb(sb):
        pltpu.async_copy(ei_hbm.at[1, sid, pl.ds(sb * SBC, SBC)], tgt_r.at[sb & 1], isem)
        pltpu.async_copy(ei_hbm.at[0, sid, pl.ds(sb * SBC, SBC)], src_r.at[sb & 1], isem)

    def drain_idx_sb(sb):
        pltpu.make_async_copy(ei_hbm.at[1, sid, pl.ds(sb * SBC, SBC)], tgt_r.at[sb & 1], isem).wait()
        pltpu.make_async_copy(ei_hbm.at[0, sid, pl.ds(sb * SBC, SBC)], src_r.at[sb & 1], isem).wait()

    def start_gathers(ck, b):
        sbi = (ck // SBC) & 1
        row = ck % SBC
        pltpu.async_copy(qt_hbm.at[cid].at[tgt_r.at[sbi, row]], qv.at[b], gsems[b])
        pltpu.async_copy(kt_hbm.at[cid].at[src_r.at[sbi, row]], kv.at[b], gsems[b])
        pltpu.async_copy(vt_hbm.at[cid].at[src_r.at[sbi, row]], vv.at[b], gsems[b])

    def drain_gathers(b):
        pltpu.make_async_copy(qt_hbm.at[cid].at[tgt_r.at[0, 0]], qv.at[b], gsems[b]).wait()
        pltpu.make_async_copy(kt_hbm.at[cid].at[src_r.at[0, 0]], kv.at[b], gsems[b]).wait()
        pltpu.make_async_copy(vt_hbm.at[cid].at[src_r.at[0, 0]], vv.at[b], gsems[b]).wait()

    def drain_scatter(b):
        pltpu.make_async_copy(wv.at[b], acc.at[tgt_r.at[0, 0]], ssems[b]).wait()

    load_idx_sb_sync(0)
    start_gathers(0, 0)

    @pl.loop(0, NSB)
    def _sb(sb):
        @pl.loop(0, SBC, step=2)
        def _pair(j0):
            for b in range(2):
                j = j0 + b
                ck = sb * SBC + j

                @pl.when(jnp.logical_and(j == 2, sb + 1 < NSB))
                def _idx_prefetch():
                    start_idx_sb(sb + 1)

                @pl.when(jnp.logical_and(j == SBC - 1, sb + 1 < NSB))
                def _idx_drain():
                    drain_idx_sb(sb + 1)

                @pl.when(ck + 1 < NCHUNK)
                def _prefetch():
                    start_gathers(ck + 1, 1 - b)

                drain_gathers(b)

                @pl.when(ck >= 2)
                def _drain_old_scatter():
                    drain_scatter(b)

                @plsc.parallel_loop(0, B, 1, unroll=2)
                def _edge(e):
                    acc16 = None
                    for t in range(DH // 32):
                        qa, qb = plsc.unpack(qv[b, e, pl.ds(t * 32, 32)],
                                             format=plsc.PackFormat.INTERLEAVED)
                        ka, kb = plsc.unpack(kv[b, e, pl.ds(t * 32, 32)],
                                             format=plsc.PackFormat.INTERLEAVED)
                        term = qa * ka + qb * kb
                        acc16 = term if acc16 is None else acc16 + term
                    t2 = acc16 + _rot(acc16, 8)
                    p = jnp.exp((t2 + _rot(t2, 4)) * SCALE)
                    wv[b, e, pl.ds(DH, 16)] = p
                    for t in range(DH // 32):
                        va, vb = plsc.unpack(vv[b, e, pl.ds(t * 32, 32)],
                                             format=plsc.PackFormat.INTERLEAVED)
                        wv[b, e, pl.ds(t * 32, 16)] = va * p
                        wv[b, e, pl.ds(t * 32 + 16, 16)] = vb * p

                pltpu.async_copy(wv.at[b], acc.at[tgt_r.at[sb & 1, j]],
                                 ssems[b], add=True)

    drain_scatter(0)
    drain_scatter(1)

    plsc.subcore_barrier()

    base = sid * ROWS_PER_TILE
    pltpu.sync_copy(acc.at[pl.ds(base, ROWS_PER_TILE)],
                    acc_hbm.at[cid, pl.ds(base, ROWS_PER_TILE)])


def _edge_stage(qt, kt, vt, ei):
    mesh = plsc.VectorSubcoreMesh(core_axis_name="c", subcore_axis_name="s")
    kern = pl.kernel(
        _edge_kernel_body,
        out_type=jax.ShapeDtypeStruct((NC, MP, WOUT), jnp.float32),
        mesh=mesh,
        scratch_types=[
            pltpu.VMEM((2, SBC, B), jnp.int32),
            pltpu.VMEM((2, SBC, B), jnp.int32),
            pltpu.VMEM((2, B, DH), jnp.bfloat16),
            pltpu.VMEM((2, B, DH), jnp.bfloat16),
            pltpu.VMEM((2, B, DH), jnp.bfloat16),
            pltpu.VMEM((2, B, WOUT), jnp.float32),
            pltpu.VMEM_SHARED((MP, WOUT), jnp.float32),
            pltpu.SemaphoreType.DMA,
            pltpu.SemaphoreType.DMA,
            pltpu.SemaphoreType.DMA,
            pltpu.SemaphoreType.DMA,
            pltpu.SemaphoreType.DMA,
        ],
        compiler_params=pltpu.CompilerParams(use_tc_tiling_on_sc=False,
                                             needs_layout_passes=False),
    )
    return kern(qt, kt, vt, ei)


def _final_body(a0_ref, a1_ref, wo_ref, bo_ref, o_ref):
    a0 = a0_ref[0]
    a1 = a1_ref[0]
    d0 = jnp.tile(a0[:, DH:DH + 4], (1, 16))
    d1 = jnp.tile(a1[:, DH:DH + 4], (1, 16))
    o0 = a0[:, :DH] / (d0 + 1e-16)
    o1 = a1[:, :DH] / (d1 + 1e-16)
    o_ref[...] = (jnp.dot(o0, wo_ref[:DH], preferred_element_type=jnp.float32)
                  + jnp.dot(o1, wo_ref[DH:], preferred_element_type=jnp.float32)
                  + bo_ref[...])


def _finalize(acc, wo, bo):
    grid = (M // _RB,)
    return pl.pallas_call(
        _final_body,
        grid=grid,
        in_specs=[
            pl.BlockSpec((1, _RB, WOUT), lambda i: (0, i, 0)),
            pl.BlockSpec((1, _RB, WOUT), lambda i: (1, i, 0)),
            pl.BlockSpec((D, D), lambda i: (0, 0)),
            pl.BlockSpec((1, D), lambda i: (0, 0)),
        ],
        out_specs=pl.BlockSpec((_RB, D), lambda i: (i, 0)),
        out_shape=jax.ShapeDtypeStruct((M, D), jnp.float32),
    )(acc, acc, wo, bo.reshape(1, D))


def kernel(s, edge_index, Wq, bq, Wkv, bkv, Wo, bo):
    w = jnp.stack([
        jnp.concatenate([Wq[:, _Q_COLS[g]], Wkv[:, _K_COLS[g]],
                         Wkv[:, _V_COLS[g]]], axis=1)
        for g in range(NC)])
    b = jnp.stack([
        jnp.concatenate([bq[_Q_COLS[g]], bkv[_K_COLS[g]], bkv[_V_COLS[g]]])
        for g in range(NC)]).reshape(NC, 1, 3 * DH)
    wo_p = Wo[_O_ROWS, :]

    qt, kt, vt = _project(s, w, b)
    ei = edge_index.reshape(2, NS, NCHUNK, B)
    acc = _edge_stage(qt, kt, vt, ei)
    return _finalize(acc, wo_p, bo)


# speedup vs baseline: 131.5111x; 1.0375x over previous
import math

import jax
import jax.numpy as jnp
import numpy as np
from jax import lax
from jax.experimental import pallas as pl
from jax.experimental.pallas import tpu as pltpu
from jax.experimental.pallas import tpu_sc as plsc

M = 10000
E = 320000
D = 128
H = 8
HD = D // H
SCALE = 1.0 / math.sqrt(HD)

NC = 2
NS = 16
EPT = E // NS
B = 125
NCHUNK = EPT // B
SBC = 10
NSB = NCHUNK // SBC
MP = 10240
ROWS_PER_TILE = MP // NS
DH = D // NC

_m = np.arange(DH)
_t, _r = _m // 32, _m % 32
_i, _e = _r // 2, _r % 2
_hl = _i % 4
_c = 8 * _t + 2 * (_i // 4) + _e


def _half_cols(g, base_of_h, off):
    h = 4 * g + _hl
    return h * base_of_h + off + _c


_Q_COLS = [_half_cols(g, HD, 0) for g in range(NC)]
_K_COLS = [_half_cols(g, 2 * HD, 0) for g in range(NC)]
_V_COLS = [_half_cols(g, 2 * HD, HD) for g in range(NC)]

_Jw = np.arange(DH)
_u, _iw = _Jw // 16, _Jw % 16
_tw, _ew = _u // 2, _u % 2
_O_ROWS = np.concatenate([
    (4 * g + _iw % 4) * HD + (8 * _tw + 2 * (_iw // 4) + _ew)
    for g in range(NC)])

_RB = 2000


def _proj_body(s_ref, w_ref, b_ref, q_ref, k_ref, v_ref):
    out = (jnp.dot(s_ref[...], w_ref[0],
                   preferred_element_type=jnp.float32) + b_ref[0])
    out = out.astype(jnp.bfloat16)
    q_ref[0] = out[:, :DH]
    k_ref[0] = out[:, DH:2 * DH]
    v_ref[0] = out[:, 2 * DH:]


def _project(s, w, b):
    grid = (NC, M // _RB)
    out_spec = pl.BlockSpec((1, _RB, DH), lambda g, i: (g, i, 0))
    return pl.pallas_call(
        _proj_body,
        grid=grid,
        in_specs=[
            pl.BlockSpec((_RB, D), lambda g, i: (i, 0)),
            pl.BlockSpec((1, D, 3 * DH), lambda g, i: (g, 0, 0)),
            pl.BlockSpec((1, 1, 3 * DH), lambda g, i: (g, 0, 0)),
        ],
        out_specs=[out_spec, out_spec, out_spec],
        out_shape=[jax.ShapeDtypeStruct((NC, M, DH), jnp.bfloat16)] * 3,
    )(s, w, b)


def _rot(x, r):
    perm = (lax.iota(jnp.int32, 16) + r) & 15
    dn = lax.GatherDimensionNumbers(
        offset_dims=(), collapsed_slice_dims=(0,), start_index_map=(0,))
    return lax.gather(x, perm[:, None], dn, (1,),
                      mode=lax.GatherScatterMode.PROMISE_IN_BOUNDS)


WOUT = DH + 16


def _edge_kernel_body(qt_hbm, kt_hbm, vt_hbm, ei_hbm, acc_hbm,
                      tgt_r, src_r, qv, kv, vv, wv, acc,
                      gsem0, gsem1, ssem0, ssem1, isem):
    cid = lax.axis_index("c")
    sid = lax.axis_index("s")
    gsems = (gsem0, gsem1)
    ssems = (ssem0, ssem1)

    @pl.loop(0, B)
    def _zfill(r):
        for t in range(WOUT // 16):
            wv[0, r, pl.ds(t * 16, 16)] = jnp.zeros((16,), jnp.float32)

    @pl.loop(0, ROWS_PER_TILE // B)
    def _zcopy(i):
        base = sid * ROWS_PER_TILE + i * B
        pltpu.sync_copy(wv.at[0], acc.at[pl.ds(base, B)])

    _ztail = ROWS_PER_TILE % B
    if _ztail:
        pltpu.sync_copy(
            wv.at[0, pl.ds(0, _ztail)],
            acc.at[pl.ds(sid * ROWS_PER_TILE + (ROWS_PER_TILE // B) * B, _ztail)])

    plsc.subcore_barrier()

    def load_idx_sb_sync(sb):
        pltpu.sync_copy(ei_hbm.at[1, sid, pl.ds(sb * SBC, SBC)], tgt_r.at[sb & 1])
        pltpu.sync_copy(ei_hbm.at[0, sid, pl.ds(sb * SBC, SBC)], src_r.at[sb & 1])

    def start_idx_sb(sb):
        pltpu.async_copy(ei_hbm.at[1, sid, pl.ds(sb * SBC, SBC)], tgt_r.at[sb & 1], isem)
        pltpu.async_copy(ei_hbm.at[0, sid, pl.ds(sb * SBC, SBC)], src_r.at[sb & 1], isem)

    def drain_idx_sb(sb):
        pltpu.make_async_copy(ei_hbm.at[1, sid, pl.ds(sb * SBC, SBC)], tgt_r.at[sb & 1], isem).wait()
        pltpu.make_async_copy(ei_hbm.at[0, sid, pl.ds(sb * SBC, SBC)], src_r.at[sb & 1], isem).wait()

    def start_gathers(ck, b):
        sbi = (ck // SBC) & 1
        row = ck % SBC
        pltpu.async_copy(qt_hbm.at[cid].at[tgt_r.at[sbi, row]], qv.at[b], gsems[b])
        pltpu.async_copy(kt_hbm.at[cid].at[src_r.at[sbi, row]], kv.at[b], gsems[b])
        pltpu.async_copy(vt_hbm.at[cid].at[src_r.at[sbi, row]], vv.at[b], gsems[b])

    def drain_gathers(b):
        pltpu.make_async_copy(qt_hbm.at[cid].at[tgt_r.at[0, 0]], qv.at[b], gsems[b]).wait()
        pltpu.make_async_copy(kt_hbm.at[cid].at[src_r.at[0, 0]], kv.at[b], gsems[b]).wait()
        pltpu.make_async_copy(vt_hbm.at[cid].at[src_r.at[0, 0]], vv.at[b], gsems[b]).wait()

    def drain_scatter(b):
        pltpu.make_async_copy(wv.at[b], acc.at[tgt_r.at[0, 0]], ssems[b]).wait()

    load_idx_sb_sync(0)
    start_gathers(0, 0)

    @pl.loop(0, NSB)
    def _sb(sb):
        @pl.loop(0, SBC, step=2)
        def _pair(j0):
            for b in range(2):
                j = j0 + b
                ck = sb * SBC + j

                @pl.when(jnp.logical_and(j == 2, sb + 1 < NSB))
                def _idx_prefetch():
                    start_idx_sb(sb + 1)

                @pl.when(jnp.logical_and(j == SBC - 1, sb + 1 < NSB))
                def _idx_drain():
                    drain_idx_sb(sb + 1)

                @pl.when(ck + 1 < NCHUNK)
                def _prefetch():
                    start_gathers(ck + 1, 1 - b)

                drain_gathers(b)

                @pl.when(ck >= 2)
                def _drain_old_scatter():
                    drain_scatter(b)

                @plsc.parallel_loop(0, B, 1, unroll=2)
                def _edge(e):
                    acc16 = None
                    for t in range(DH // 32):
                        qa, qb = plsc.unpack(qv[b, e, pl.ds(t * 32, 32)],
                                             format=plsc.PackFormat.INTERLEAVED)
                        ka, kb = plsc.unpack(kv[b, e, pl.ds(t * 32, 32)],
                                             format=plsc.PackFormat.INTERLEAVED)
                        term = qa * ka + qb * kb
                        acc16 = term if acc16 is None else acc16 + term
                    t2 = acc16 + _rot(acc16, 8)
                    p = jnp.exp((t2 + _rot(t2, 4)) * SCALE)
                    wv[b, e, pl.ds(DH, 16)] = p
                    for t in range(DH // 32):
                        va, vb = plsc.unpack(vv[b, e, pl.ds(t * 32, 32)],
                                             format=plsc.PackFormat.INTERLEAVED)
                        wv[b, e, pl.ds(t * 32, 16)] = va * p
                        wv[b, e, pl.ds(t * 32 + 16, 16)] = vb * p

                pltpu.async_copy(wv.at[b], acc.at[tgt_r.at[sb & 1, j]],
                                 ssems[b], add=True)

    drain_scatter(0)
    drain_scatter(1)

    plsc.subcore_barrier()

    base = sid * ROWS_PER_TILE
    pltpu.sync_copy(acc.at[pl.ds(base, ROWS_PER_TILE)],
                    acc_hbm.at[cid, pl.ds(base, ROWS_PER_TILE)])


def _edge_stage(qt, kt, vt, ei):
    mesh = plsc.VectorSubcoreMesh(core_axis_name="c", subcore_axis_name="s")
    kern = pl.kernel(
        _edge_kernel_body,
        out_type=jax.ShapeDtypeStruct((NC, MP, WOUT), jnp.float32),
        mesh=mesh,
        scratch_types=[
            pltpu.VMEM((2, SBC, B), jnp.int32),
            pltpu.VMEM((2, SBC, B), jnp.int32),
            pltpu.VMEM((2, B, DH), jnp.bfloat16),
            pltpu.VMEM((2, B, DH), jnp.bfloat16),
            pltpu.VMEM((2, B, DH), jnp.bfloat16),
            pltpu.VMEM((2, B, WOUT), jnp.float32),
            pltpu.VMEM_SHARED((MP, WOUT), jnp.float32),
            pltpu.SemaphoreType.DMA,
            pltpu.SemaphoreType.DMA,
            pltpu.SemaphoreType.DMA,
            pltpu.SemaphoreType.DMA,
            pltpu.SemaphoreType.DMA,
        ],
        compiler_params=pltpu.CompilerParams(use_tc_tiling_on_sc=False,
                                             needs_layout_passes=False),
    )
    return kern(qt, kt, vt, ei)


def _final_body(a0_ref, a1_ref, wo_ref, bo_ref, o_ref):
    a0 = a0_ref[0]
    a1 = a1_ref[0]
    d0 = jnp.tile(a0[:, DH:DH + 4], (1, 16))
    d1 = jnp.tile(a1[:, DH:DH + 4], (1, 16))
    o0 = a0[:, :DH] / (d0 + 1e-16)
    o1 = a1[:, :DH] / (d1 + 1e-16)
    o_ref[...] = (jnp.dot(o0, wo_ref[:DH], preferred_element_type=jnp.float32)
                  + jnp.dot(o1, wo_ref[DH:], preferred_element_type=jnp.float32)
                  + bo_ref[...])


def _finalize(acc, wo, bo):
    grid = (M // _RB,)
    return pl.pallas_call(
        _final_body,
        grid=grid,
        in_specs=[
            pl.BlockSpec((1, _RB, WOUT), lambda i: (0, i, 0)),
            pl.BlockSpec((1, _RB, WOUT), lambda i: (1, i, 0)),
            pl.BlockSpec((D, D), lambda i: (0, 0)),
            pl.BlockSpec((1, D), lambda i: (0, 0)),
        ],
        out_specs=pl.BlockSpec((_RB, D), lambda i: (i, 0)),
        out_shape=jax.ShapeDtypeStruct((M, D), jnp.float32),
    )(acc, acc, wo, bo.reshape(1, D))


def kernel(s, edge_index, Wq, bq, Wkv, bkv, Wo, bo):
    w = jnp.stack([
        jnp.concatenate([Wq[:, _Q_COLS[g]], Wkv[:, _K_COLS[g]],
                         Wkv[:, _V_COLS[g]]], axis=1)
        for g in range(NC)])
    b = jnp.stack([
        jnp.concatenate([bq[_Q_COLS[g]], bkv[_K_COLS[g]], bkv[_V_COLS[g]]])
        for g in range(NC)]).reshape(NC, 1, 3 * DH)
    wo_p = Wo[_O_ROWS, :]

    qt, kt, vt = _project(s, w, b)
    ei = edge_index.reshape(2, NS, NCHUNK, B)
    acc = _edge_stage(qt, kt, vt, ei)
    return _finalize(acc, wo_p, bo)
